# baseline (speedup 1.0000x reference)
import functools

import jax
import jax.numpy as jnp
from jax.experimental import pallas as pl
from jax.experimental.pallas import tpu as pltpu

ALPHA = 0.25
GAMMA_POW = 2


def _body(cls_ref, reg_ref, anc_ref, box_ref, lab_ref, out_ref, *, A, TA, T, C, G):
    t = pl.program_id(1)

    anc = anc_ref[...]
    box = box_ref[0]
    lab = lab_ref[0]

    a0 = anc[0:1, :]
    a1 = anc[1:2, :]
    a2 = anc[2:3, :]
    a3 = anc[3:4, :]

    b0 = box[:, 0:1]
    b1 = box[:, 1:2]
    b2 = box[:, 2:3]
    b3 = box[:, 3:4]

    area_b = (b2 - b0) * (b3 - b1)
    iw = jnp.minimum(a2, b2) - jnp.maximum(a0, b0)
    ih = jnp.minimum(a3, b3) - jnp.maximum(a1, b1)
    iw = jnp.clip(iw, 0.0, None)
    ih = jnp.clip(ih, 0.0, None)
    area_a = (a2 - a0) * (a3 - a1)
    inter = iw * ih
    ua = area_a + area_b - inter
    iou = inter / ua

    iou_max = jnp.max(iou, axis=0, keepdims=True)
    g_iota = jax.lax.broadcasted_iota(jnp.int32, (G, TA), 0)
    amax = jnp.min(jnp.where(iou == iou_max, g_iota, G), axis=0, keepdims=True)
    onehot = g_iota == amax

    a_idx = t * TA + jax.lax.broadcasted_iota(jnp.int32, (1, TA), 1)
    inb = a_idx < A
    pos = jnp.logical_and(iou_max >= 0.5, inb)
    valid = jnp.logical_and(
        jnp.logical_or(iou_max < 0.4, iou_max >= 0.5), inb)

    ponehot = jnp.where(jnp.logical_and(onehot, pos), 1.0, 0.0)

    p = b2 - b0
    q = b3 - b1
    r = 0.5 * (b0 + b2)
    s = 0.5 * (b1 + b3)
    boxmix = jnp.concatenate([p, q, r, s], axis=1)
    oh_bf = ponehot.astype(jnp.bfloat16)
    dn_g = (((0,), (0,)), ((), ()))
    bm1 = boxmix.astype(jnp.bfloat16)
    bmr1 = boxmix - bm1.astype(jnp.float32)
    bm2 = bmr1.astype(jnp.bfloat16)
    bm3 = (bmr1 - bm2.astype(jnp.float32)).astype(jnp.bfloat16)
    asg = (jax.lax.dot_general(bm1, oh_bf, dn_g,
                               preferred_element_type=jnp.float32)
           + jax.lax.dot_general(bm2, oh_bf, dn_g,
                                 preferred_element_type=jnp.float32)
           + jax.lax.dot_general(bm3, oh_bf, dn_g,
                                 preferred_element_type=jnp.float32))
    gw0 = asg[0:1, :]
    gh0 = asg[1:2, :]
    gcx = asg[2:3, :]
    gcy = asg[3:4, :]

    a_w = a2 - a0
    a_h = a3 - a1
    a_cx = a0 + 0.5 * a_w
    a_cy = a1 + 0.5 * a_h
    gw = jnp.clip(gw0, 1.0, None)
    gh = jnp.clip(gh0, 1.0, None)
    t0 = ((gcx - a_cx) / a_w) / 0.1
    t1 = ((gcy - a_cy) / a_h) / 0.1
    t2 = jnp.log(gw / a_w) / 0.2
    t3 = jnp.log(gh / a_h) / 0.2

    reg = reg_ref[0]
    regacc = jnp.zeros((1, TA), jnp.float32)
    for k, tk in enumerate((t0, t1, t2, t3)):
        d = jnp.abs(tk - reg[k:k + 1, :])
        rl = jnp.where(d <= 1.0 / 9.0, 0.5 * 9.0 * d * d, d - 0.5 / 9.0)
        regacc = regacc + jnp.where(pos, rl, 0.0)

    craw = cls_ref[0]
    cls = jnp.where(craw > 1e-4, craw, 1e-4)
    cls = jnp.where(cls < 1.0 - 1e-4, cls, 1.0 - 1e-4)
    om = 1.0 - cls
    cls_bf = cls.astype(jnp.bfloat16)
    om_bf = om.astype(jnp.bfloat16)
    nlom_bf = (-jnp.log(om)).astype(jnp.bfloat16)
    neg_bf = (jnp.bfloat16(1.0 - ALPHA) * cls_bf * cls_bf
              * nlom_bf)

    validf_bf = jnp.where(valid, 1.0, 0.0).astype(jnp.bfloat16)
    sneg_vec = jax.lax.dot_general(validf_bf, neg_bf, (((1,), (0,)), ((), ())),
                                   preferred_element_type=jnp.float32)

    pos_bf = (jnp.bfloat16(ALPHA) * om_bf * om_bf
              * (-jnp.log(cls_bf)))
    delta_bf = pos_bf - neg_bf
    ponehot_bf = ponehot.astype(jnp.bfloat16)
    D = jax.lax.dot_general(ponehot_bf, delta_bf, (((1,), (0,)), ((), ())),
                            preferred_element_type=jnp.float32)
    c_iota_g = jax.lax.broadcasted_iota(jnp.int32, (G, C), 1)
    Lf = jnp.where(c_iota_g == lab, 1.0, 0.0)
    corr_vec = jnp.sum(Lf * D, axis=0, keepdims=True)

    nposv = jnp.where(pos, 1.0, 0.0)

    def fold128(v):
        acc = v[:, 0:128]
        for kk in range(1, TA // 128):
            acc = acc + v[:, kk * 128:(kk + 1) * 128]
        return acc

    cls_row = jnp.concatenate(
        [sneg_vec + corr_vec, jnp.zeros((1, 128 - C), jnp.float32)], axis=1)
    vec = jnp.concatenate(
        [cls_row, jnp.zeros((1, 128), jnp.float32), fold128(regacc),
         fold128(nposv)], axis=0)[None]

    @pl.when(t == 0)
    def _init():
        out_ref[...] = vec

    @pl.when(t > 0)
    def _acc():
        out_ref[...] = out_ref[...] + vec

    @pl.when(t == T - 1)
    def _fin():
        acc = out_ref[...]
        cls_sum = jnp.sum(acc[0, 0:2, :])
        reg_sum = jnp.sum(acc[0, 2:3, :])
        npos = jnp.maximum(jnp.sum(acc[0, 3:4, :]), 1.0)
        cls_l = cls_sum / npos
        reg_l = reg_sum / (npos * 4.0)
        l_iota = jax.lax.broadcasted_iota(jnp.int32, (1, 4, 128), 2)
        r_iota = jax.lax.broadcasted_iota(jnp.int32, (1, 4, 128), 1)
        out_ref[...] = (
            jnp.where(jnp.logical_and(r_iota == 0, l_iota == 0), cls_l, 0.0)
            + jnp.where(jnp.logical_and(r_iota == 0, l_iota == 1), reg_l, 0.0))


def _transpose_last2(x):
    K = x.shape[2]
    eye2 = 2.0 * jnp.eye(K, dtype=x.dtype)
    y = jnp.einsum('kc,bac->bka', eye2, x,
                   precision=jax.lax.Precision.HIGHEST)
    return 0.5 * y


@jax.jit
def kernel(classifications, regressions, anchors, boxes, labels):
    B, A, C = classifications.shape
    G = boxes.shape[1]
    TA = 4096
    T = (A + TA - 1) // TA

    reg_t = _transpose_last2(regressions)
    anc_t = _transpose_last2(anchors)[0]
    lab3 = labels.astype(jnp.int32)[..., None]

    body = functools.partial(_body, A=A, TA=TA, T=T, C=C, G=G)
    out = pl.pallas_call(
        body,
        grid=(B, T),
        in_specs=[
            pl.BlockSpec((1, TA, C), lambda j, t: (j, t, 0)),
            pl.BlockSpec((1, 4, TA), lambda j, t: (j, 0, t)),
            pl.BlockSpec((4, TA), lambda j, t: (0, t)),
            pl.BlockSpec((1, G, 4), lambda j, t: (j, 0, 0)),
            pl.BlockSpec((1, G, 1), lambda j, t: (j, 0, 0)),
        ],
        out_specs=pl.BlockSpec((1, 4, 128), lambda j, t: (j, 0, 0)),
        out_shape=jax.ShapeDtypeStruct((B, 4, 128), jnp.float32),
        compiler_params=pltpu.CompilerParams(
            dimension_semantics=("arbitrary", "arbitrary")),
    )(classifications, reg_t, anc_t, boxes, lab3)

    cls_loss = jnp.mean(out[:, 0, 0:1], axis=0)
    reg_loss = jnp.mean(out[:, 0, 1:2], axis=0)
    return cls_loss, reg_loss


# speedup vs baseline: 1.8026x; 1.8026x over previous
import functools

import jax
import jax.numpy as jnp
from jax.experimental import pallas as pl
from jax.experimental.pallas import tpu as pltpu

ALPHA = 0.25
GAMMA_POW = 2


def _body(cls_ref, reg_ref, anc_ref, box_ref, lab_ref, out_ref, *, A, TA, T, C, G):
    t = pl.program_id(1)

    anc = anc_ref[...]
    box = box_ref[0]
    lab = lab_ref[0]

    a0 = anc[0:1, :]
    a1 = anc[1:2, :]
    a2 = anc[2:3, :]
    a3 = anc[3:4, :]

    b0 = box[:, 0:1]
    b1 = box[:, 1:2]
    b2 = box[:, 2:3]
    b3 = box[:, 3:4]

    area_b = (b2 - b0) * (b3 - b1)
    iw = jnp.minimum(a2, b2) - jnp.maximum(a0, b0)
    ih = jnp.minimum(a3, b3) - jnp.maximum(a1, b1)
    iw = jnp.clip(iw, 0.0, None)
    ih = jnp.clip(ih, 0.0, None)
    area_a = (a2 - a0) * (a3 - a1)
    inter = iw * ih
    ua = area_a + area_b - inter
    iou = inter / ua

    iou_max = jnp.max(iou, axis=0, keepdims=True)
    g_col = jax.lax.broadcasted_iota(jnp.int32, (G, 1), 0)
    amax = jnp.min(jnp.where(iou == iou_max, g_col, G), axis=0, keepdims=True)
    onehot = g_col == amax

    a_idx = t * TA + jax.lax.broadcasted_iota(jnp.int32, (1, TA), 1)
    inb = a_idx < A
    pos = jnp.logical_and(iou_max >= 0.5, inb)
    valid = jnp.logical_and(
        jnp.logical_or(iou_max < 0.4, iou_max >= 0.5), inb)

    ponehot = jnp.where(jnp.logical_and(onehot, pos), 1.0, 0.0)

    p = b2 - b0
    q = b3 - b1
    r = 0.5 * (b0 + b2)
    s = 0.5 * (b1 + b3)
    boxmix = jnp.concatenate([p, q, r, s], axis=1)
    oh_bf = ponehot.astype(jnp.bfloat16)
    dn_g = (((0,), (0,)), ((), ()))
    bm1 = boxmix.astype(jnp.bfloat16)
    bmr1 = boxmix - bm1.astype(jnp.float32)
    bm2 = bmr1.astype(jnp.bfloat16)
    bm3 = (bmr1 - bm2.astype(jnp.float32)).astype(jnp.bfloat16)
    asg = (jax.lax.dot_general(bm1, oh_bf, dn_g,
                               preferred_element_type=jnp.float32)
           + jax.lax.dot_general(bm2, oh_bf, dn_g,
                                 preferred_element_type=jnp.float32)
           + jax.lax.dot_general(bm3, oh_bf, dn_g,
                                 preferred_element_type=jnp.float32))
    gw0 = asg[0:1, :]
    gh0 = asg[1:2, :]
    gcx = asg[2:3, :]
    gcy = asg[3:4, :]

    a_w = a2 - a0
    a_h = a3 - a1
    a_cx = a0 + 0.5 * a_w
    a_cy = a1 + 0.5 * a_h
    gw = jnp.clip(gw0, 1.0, None)
    gh = jnp.clip(gh0, 1.0, None)
    t0 = ((gcx - a_cx) / a_w) / 0.1
    t1 = ((gcy - a_cy) / a_h) / 0.1
    t2 = jnp.log(gw / a_w) / 0.2
    t3 = jnp.log(gh / a_h) / 0.2

    reg = reg_ref[0]
    regacc = jnp.zeros((1, TA), jnp.float32)
    for k, tk in enumerate((t0, t1, t2, t3)):
        d = jnp.abs(tk - reg[k:k + 1, :])
        rl = jnp.where(d <= 1.0 / 9.0, 0.5 * 9.0 * d * d, d - 0.5 / 9.0)
        regacc = regacc + jnp.where(pos, rl, 0.0)

    craw = cls_ref[0]
    cls = jnp.where(craw > 1e-4, craw, 1e-4)
    cls = jnp.where(cls < 1.0 - 1e-4, cls, 1.0 - 1e-4)
    om = 1.0 - cls
    cls_bf = cls.astype(jnp.bfloat16)
    om_bf = om.astype(jnp.bfloat16)
    lom_bf = jnp.log(om).astype(jnp.bfloat16)
    neg_bf = (jnp.bfloat16(-(1.0 - ALPHA)) * cls_bf * cls_bf
              * lom_bf)

    validf_bf = jnp.where(valid, 1.0, 0.0).astype(jnp.bfloat16)
    sneg_vec = jax.lax.dot_general(validf_bf, neg_bf, (((1,), (1,)), ((), ())),
                                   preferred_element_type=jnp.float32)

    pos_bf = (jnp.bfloat16(-ALPHA) * om_bf * om_bf
              * jnp.log(cls_bf))
    delta_bf = pos_bf - neg_bf
    ponehot_bf = ponehot.astype(jnp.bfloat16)
    D = jax.lax.dot_general(ponehot_bf, delta_bf, (((1,), (1,)), ((), ())),
                            preferred_element_type=jnp.float32)
    c_iota_g = jax.lax.broadcasted_iota(jnp.int32, (G, C), 1)
    Lf = jnp.where(c_iota_g == lab, 1.0, 0.0)
    corr_vec = jnp.sum(Lf * D, axis=0, keepdims=True)

    nposv = jnp.where(pos, 1.0, 0.0)

    def fold128(v):
        acc = v[:, 0:128]
        for kk in range(1, TA // 128):
            acc = acc + v[:, kk * 128:(kk + 1) * 128]
        return acc

    cls_row = jnp.concatenate(
        [sneg_vec + corr_vec, jnp.zeros((1, 128 - C), jnp.float32)], axis=1)
    vec = jnp.concatenate(
        [cls_row, jnp.zeros((1, 128), jnp.float32), fold128(regacc),
         fold128(nposv)], axis=0)[None]

    @pl.when(t == 0)
    def _init():
        out_ref[...] = vec

    @pl.when(t > 0)
    def _acc():
        out_ref[...] = out_ref[...] + vec

    @pl.when(t == T - 1)
    def _fin():
        acc = out_ref[...]
        cls_sum = jnp.sum(acc[0, 0:2, :])
        reg_sum = jnp.sum(acc[0, 2:3, :])
        npos = jnp.maximum(jnp.sum(acc[0, 3:4, :]), 1.0)
        cls_l = cls_sum / npos
        reg_l = reg_sum / (npos * 4.0)
        l_iota = jax.lax.broadcasted_iota(jnp.int32, (1, 4, 128), 2)
        r_iota = jax.lax.broadcasted_iota(jnp.int32, (1, 4, 128), 1)
        out_ref[...] = (
            jnp.where(jnp.logical_and(r_iota == 0, l_iota == 0), cls_l, 0.0)
            + jnp.where(jnp.logical_and(r_iota == 0, l_iota == 1), reg_l, 0.0))


@jax.jit
def kernel(classifications, regressions, anchors, boxes, labels):
    B, A, C = classifications.shape
    G = boxes.shape[1]
    TA = 8192
    T = (A + TA - 1) // TA

    cls_t = jnp.transpose(classifications, (0, 2, 1))
    reg_t = jnp.transpose(regressions, (0, 2, 1))
    anc_t = jnp.transpose(anchors, (0, 2, 1))[0]
    lab3 = labels.astype(jnp.int32)[..., None]

    body = functools.partial(_body, A=A, TA=TA, T=T, C=C, G=G)
    out = pl.pallas_call(
        body,
        grid=(B, T),
        in_specs=[
            pl.BlockSpec((1, C, TA), lambda j, t: (j, 0, t)),
            pl.BlockSpec((1, 4, TA), lambda j, t: (j, 0, t)),
            pl.BlockSpec((4, TA), lambda j, t: (0, t)),
            pl.BlockSpec((1, G, 4), lambda j, t: (j, 0, 0)),
            pl.BlockSpec((1, G, 1), lambda j, t: (j, 0, 0)),
        ],
        out_specs=pl.BlockSpec((1, 4, 128), lambda j, t: (j, 0, 0)),
        out_shape=jax.ShapeDtypeStruct((B, 4, 128), jnp.float32),
        compiler_params=pltpu.CompilerParams(
            dimension_semantics=("arbitrary", "arbitrary")),
    )(cls_t, reg_t, anc_t, boxes, lab3)

    cls_loss = jnp.mean(out[:, 0, 0:1], axis=0)
    reg_loss = jnp.mean(out[:, 0, 1:2], axis=0)
    return cls_loss, reg_loss


# speedup vs baseline: 1.8770x; 1.0413x over previous
import functools

import jax
import jax.numpy as jnp
from jax.experimental import pallas as pl
from jax.experimental.pallas import tpu as pltpu

ALPHA = 0.25
GAMMA_POW = 2


def _body(cls_ref, reg_ref, anc_ref, box_ref, lab_ref, out_ref, *, A, TA, T, C, G):
    t = pl.program_id(1)

    anc = anc_ref[...]
    box = box_ref[0]
    lab = lab_ref[0]

    a0 = anc[0:1, :]
    a1 = anc[1:2, :]
    a2 = anc[2:3, :]
    a3 = anc[3:4, :]

    b0 = box[:, 0:1]
    b1 = box[:, 1:2]
    b2 = box[:, 2:3]
    b3 = box[:, 3:4]

    area_b = (b2 - b0) * (b3 - b1)
    iw = jnp.minimum(a2, b2) - jnp.maximum(a0, b0)
    ih = jnp.minimum(a3, b3) - jnp.maximum(a1, b1)
    iw = jnp.clip(iw, 0.0, None)
    ih = jnp.clip(ih, 0.0, None)
    area_a = (a2 - a0) * (a3 - a1)
    inter = iw * ih
    ua = area_a + area_b - inter
    iou = inter / ua

    iou_max = jnp.max(iou, axis=0, keepdims=True)
    g_col = jax.lax.broadcasted_iota(jnp.int32, (G, 1), 0)
    amax = jnp.min(jnp.where(iou == iou_max, g_col, G), axis=0, keepdims=True)
    onehot = g_col == amax

    a_idx = t * TA + jax.lax.broadcasted_iota(jnp.int32, (1, TA), 1)
    inb = a_idx < A
    pos = jnp.logical_and(iou_max >= 0.5, inb)
    valid = jnp.logical_and(
        jnp.logical_or(iou_max < 0.4, iou_max >= 0.5), inb)

    ponehot = jnp.where(jnp.logical_and(onehot, pos), 1.0, 0.0)

    p = b2 - b0
    q = b3 - b1
    r = 0.5 * (b0 + b2)
    s = 0.5 * (b1 + b3)
    boxmix = jnp.concatenate([p, q, r, s], axis=1)
    oh_bf = ponehot.astype(jnp.bfloat16)
    dn_g = (((0,), (0,)), ((), ()))
    bm1 = boxmix.astype(jnp.bfloat16)
    bmr1 = boxmix - bm1.astype(jnp.float32)
    bm2 = bmr1.astype(jnp.bfloat16)
    bm3 = (bmr1 - bm2.astype(jnp.float32)).astype(jnp.bfloat16)
    bms = jnp.concatenate([bm1, bm2, bm3], axis=1)
    asg3 = jax.lax.dot_general(bms, oh_bf, dn_g,
                               preferred_element_type=jnp.float32)
    asg = asg3[0:4, :] + asg3[4:8, :] + asg3[8:12, :]
    gw0 = asg[0:1, :]
    gh0 = asg[1:2, :]
    gcx = asg[2:3, :]
    gcy = asg[3:4, :]

    a_w = a2 - a0
    a_h = a3 - a1
    a_cx = a0 + 0.5 * a_w
    a_cy = a1 + 0.5 * a_h
    gw = jnp.clip(gw0, 1.0, None)
    gh = jnp.clip(gh0, 1.0, None)
    t0 = ((gcx - a_cx) / a_w) / 0.1
    t1 = ((gcy - a_cy) / a_h) / 0.1
    t2 = jnp.log(gw / a_w) / 0.2
    t3 = jnp.log(gh / a_h) / 0.2

    reg = reg_ref[0]
    regacc = jnp.zeros((1, TA), jnp.float32)
    for k, tk in enumerate((t0, t1, t2, t3)):
        d = jnp.abs(tk - reg[k:k + 1, :])
        rl = jnp.where(d <= 1.0 / 9.0, 0.5 * 9.0 * d * d, d - 0.5 / 9.0)
        regacc = regacc + jnp.where(pos, rl, 0.0)

    craw = cls_ref[0]
    cls = jnp.where(craw > 1e-4, craw, 1e-4)
    cls = jnp.where(cls < 1.0 - 1e-4, cls, 1.0 - 1e-4)
    om = 1.0 - cls
    cls_bf = cls.astype(jnp.bfloat16)
    om_bf = om.astype(jnp.bfloat16)
    lom_bf = jnp.log(om).astype(jnp.bfloat16)
    neg_bf = (jnp.bfloat16(-(1.0 - ALPHA)) * cls_bf * cls_bf
              * lom_bf)

    validf_bf = jnp.where(valid, 1.0, 0.0).astype(jnp.bfloat16)
    sneg_vec = jax.lax.dot_general(validf_bf, neg_bf, (((1,), (1,)), ((), ())),
                                   preferred_element_type=jnp.float32)

    pos_bf = (jnp.bfloat16(-ALPHA) * om_bf * om_bf
              * jnp.log(cls_bf))
    delta_bf = pos_bf - neg_bf
    ponehot_bf = ponehot.astype(jnp.bfloat16)
    D = jax.lax.dot_general(ponehot_bf, delta_bf, (((1,), (1,)), ((), ())),
                            preferred_element_type=jnp.float32)
    c_iota_g = jax.lax.broadcasted_iota(jnp.int32, (G, C), 1)
    Lf = jnp.where(c_iota_g == lab, 1.0, 0.0)
    corr_vec = jnp.sum(Lf * D, axis=0, keepdims=True)

    nposv = jnp.where(pos, 1.0, 0.0)

    def fold128(v):
        acc = v[:, 0:128]
        for kk in range(1, TA // 128):
            acc = acc + v[:, kk * 128:(kk + 1) * 128]
        return acc

    cls_row = jnp.concatenate(
        [sneg_vec + corr_vec, jnp.zeros((1, 128 - C), jnp.float32)], axis=1)
    vec = jnp.concatenate(
        [cls_row, jnp.zeros((1, 128), jnp.float32), fold128(regacc),
         fold128(nposv)], axis=0)[None]

    @pl.when(t == 0)
    def _init():
        out_ref[...] = vec

    @pl.when(t > 0)
    def _acc():
        out_ref[...] = out_ref[...] + vec

    @pl.when(t == T - 1)
    def _fin():
        acc = out_ref[...]
        cls_sum = jnp.sum(acc[0, 0:2, :])
        reg_sum = jnp.sum(acc[0, 2:3, :])
        npos = jnp.maximum(jnp.sum(acc[0, 3:4, :]), 1.0)
        cls_l = cls_sum / npos
        reg_l = reg_sum / (npos * 4.0)
        l_iota = jax.lax.broadcasted_iota(jnp.int32, (1, 4, 128), 2)
        r_iota = jax.lax.broadcasted_iota(jnp.int32, (1, 4, 128), 1)
        out_ref[...] = (
            jnp.where(jnp.logical_and(r_iota == 0, l_iota == 0), cls_l, 0.0)
            + jnp.where(jnp.logical_and(r_iota == 0, l_iota == 1), reg_l, 0.0))


@jax.jit
def kernel(classifications, regressions, anchors, boxes, labels):
    B, A, C = classifications.shape
    G = boxes.shape[1]
    TA = 8192
    T = (A + TA - 1) // TA

    cls_t = jnp.transpose(classifications, (0, 2, 1))
    reg_t = jnp.transpose(regressions, (0, 2, 1))
    anc_t = jnp.transpose(anchors, (0, 2, 1))[0]
    lab3 = labels.astype(jnp.int32)[..., None]

    body = functools.partial(_body, A=A, TA=TA, T=T, C=C, G=G)
    out = pl.pallas_call(
        body,
        grid=(B, T),
        in_specs=[
            pl.BlockSpec((1, C, TA), lambda j, t: (j, 0, t)),
            pl.BlockSpec((1, 4, TA), lambda j, t: (j, 0, t)),
            pl.BlockSpec((4, TA), lambda j, t: (0, t)),
            pl.BlockSpec((1, G, 4), lambda j, t: (j, 0, 0)),
            pl.BlockSpec((1, G, 1), lambda j, t: (j, 0, 0)),
        ],
        out_specs=pl.BlockSpec((1, 4, 128), lambda j, t: (j, 0, 0)),
        out_shape=jax.ShapeDtypeStruct((B, 4, 128), jnp.float32),
        compiler_params=pltpu.CompilerParams(
            dimension_semantics=("arbitrary", "arbitrary")),
    )(cls_t, reg_t, anc_t, boxes, lab3)

    cls_loss = jnp.mean(out[:, 0, 0:1], axis=0)
    reg_loss = jnp.mean(out[:, 0, 1:2], axis=0)
    return cls_loss, reg_loss
